# initial kernel scaffold (unmeasured)
import functools

import jax
import jax.numpy as jnp
from jax import lax
from jax.experimental import pallas as pl
from jax.experimental.pallas import tpu as pltpu

N_DEV = 8
B_PER = 2
SQ = 256
H_PER = 4
DH = 64
DM = 512
DQ = H_PER * DH


def kernel(x, Wq, K_ext, V_ext, Wo):
    p = lax.axis_index("i")

    k_l = lax.dynamic_slice(K_ext, (0, 0, 4 * p, 0), (16, SQ, H_PER, DH))
    v_l = lax.dynamic_slice(V_ext, (0, 0, 4 * p, 0), (16, SQ, H_PER, DH))
    x16 = x.astype(jnp.bfloat16)
    wq16 = Wq.astype(jnp.bfloat16)
    wo16 = Wo.astype(jnp.bfloat16)
    k16 = k_l.astype(jnp.bfloat16)
    v16 = v_l.astype(jnp.bfloat16)

    def body(x_ref, wq_ref, k_ref, v_ref, wo_ref, out_ref,
             xall, part, rs_buf,
             ag_send_sems, ag_recv_sems, rs_send_sems, rs_recv_sems):
        my = lax.axis_index("i")
        left = lax.rem(my - 1 + N_DEV, N_DEV)
        right = lax.rem(my + 1, N_DEV)

        barrier = pltpu.get_barrier_semaphore()
        pl.semaphore_signal(barrier, inc=1, device_id=(left,),
                            device_id_type=pl.DeviceIdType.MESH)
        pl.semaphore_signal(barrier, inc=1, device_id=(right,),
                            device_id_type=pl.DeviceIdType.MESH)
        pl.semaphore_wait(barrier, 2)

        xall[my] = x_ref[...]
        for h in range(N_DEV - 1):
            src_chunk = lax.rem(my - h + N_DEV, N_DEV)
            dst_chunk = src_chunk
            rdma = pltpu.make_async_remote_copy(
                src_ref=xall.at[src_chunk],
                dst_ref=xall.at[dst_chunk],
                send_sem=ag_send_sems.at[h],
                recv_sem=ag_recv_sems.at[h],
                device_id=(right,),
                device_id_type=pl.DeviceIdType.MESH,
            )
            rdma.start()
            rdma.wait()

        X = xall[...].reshape(N_DEV * B_PER * SQ, DM)
        Q = jnp.dot(X, wq_ref[...],
                    preferred_element_type=jnp.bfloat16)
        Qr = Q.reshape(16, SQ, H_PER, DH)

        scores = lax.dot_general(
            Qr, k_ref[...],
            dimension_numbers=(((3,), (3,)), ((0, 2), (0, 2))),
            preferred_element_type=jnp.float32,
        ) * 0.125
        qi = lax.broadcasted_iota(jnp.int32, (SQ, SQ), 0) // 64
        kj = lax.broadcasted_iota(jnp.int32, (SQ, SQ), 1) // 64
        mask = (qi == kj)[None, None, :, :]
        scores = jnp.where(mask, scores, -1e9)
        m = jnp.max(scores, axis=-1, keepdims=True)
        w = jnp.exp(scores - m)
        w = w / jnp.sum(w, axis=-1, keepdims=True)
        wb = w.astype(jnp.bfloat16)

        ctx = lax.dot_general(
            wb, v_ref[...],
            dimension_numbers=(((3,), (1,)), ((0, 1), (0, 2))),
            preferred_element_type=jnp.bfloat16,
        )
        ctxt = jnp.transpose(ctx, (0, 2, 1, 3)).reshape(16 * SQ, DQ)
        partial = jnp.dot(ctxt, wo_ref[...],
                          preferred_element_type=jnp.float32)
        part[...] = partial.reshape(N_DEV, B_PER, SQ, DM)

        first = lax.rem(my - 1 + N_DEV, N_DEV)
        rs_buf[0] = part[first]
        for t in range(N_DEV - 1):
            rdma = pltpu.make_async_remote_copy(
                src_ref=rs_buf.at[t],
                dst_ref=rs_buf.at[t + 1],
                send_sem=rs_send_sems.at[t],
                recv_sem=rs_recv_sems.at[t],
                device_id=(right,),
                device_id_type=pl.DeviceIdType.MESH,
            )
            rdma.start()
            rdma.wait()
            chunk = lax.rem(my - 2 - t + N_DEV, N_DEV)
            if t < N_DEV - 2:
                rs_buf[t + 1] = rs_buf[t + 1] + part[chunk]
            else:
                out_ref[...] = rs_buf[t + 1] + part[chunk]

    grid_spec = pltpu.PrefetchScalarGridSpec(
        num_scalar_prefetch=0,
        in_specs=[
            pl.BlockSpec(memory_space=pltpu.VMEM),
            pl.BlockSpec(memory_space=pltpu.VMEM),
            pl.BlockSpec(memory_space=pltpu.VMEM),
            pl.BlockSpec(memory_space=pltpu.VMEM),
            pl.BlockSpec(memory_space=pltpu.VMEM),
        ],
        out_specs=pl.BlockSpec(memory_space=pltpu.VMEM),
        scratch_shapes=[
            pltpu.VMEM((N_DEV, B_PER, SQ, DM), jnp.bfloat16),
            pltpu.VMEM((N_DEV, B_PER, SQ, DM), jnp.float32),
            pltpu.VMEM((N_DEV, B_PER, SQ, DM), jnp.float32),
            pltpu.SemaphoreType.DMA((N_DEV - 1,)),
            pltpu.SemaphoreType.DMA((N_DEV - 1,)),
            pltpu.SemaphoreType.DMA((N_DEV - 1,)),
            pltpu.SemaphoreType.DMA((N_DEV - 1,)),
        ],
    )

    return pl.pallas_call(
        body,
        out_shape=jax.ShapeDtypeStruct((B_PER, SQ, DM), jnp.float32),
        grid_spec=grid_spec,
        compiler_params=pltpu.CompilerParams(collective_id=0),
    )(x16, wq16, k16, v16, wo16)


# baseline (device time: 183034 ns/iter reference)
import functools

import jax
import jax.numpy as jnp
from jax import lax
from jax.experimental import pallas as pl
from jax.experimental.pallas import tpu as pltpu

N_DEV = 8
B_PER = 2
SQ = 256
H_PER = 4
DH = 64
DM = 512
DQ = H_PER * DH


def kernel(x, Wq, K_ext, V_ext, Wo):
    p = lax.axis_index("i")

    k_l = lax.dynamic_slice(K_ext, (0, 0, 4 * p, 0), (16, SQ, H_PER, DH))
    v_l = lax.dynamic_slice(V_ext, (0, 0, 4 * p, 0), (16, SQ, H_PER, DH))
    x16 = x.astype(jnp.bfloat16)
    wq16 = Wq.astype(jnp.bfloat16)
    wo16 = Wo.astype(jnp.bfloat16)
    k16 = jnp.transpose(k_l, (2, 0, 1, 3)).astype(jnp.bfloat16)
    v16 = jnp.transpose(v_l, (2, 0, 1, 3)).astype(jnp.bfloat16)

    def body(x_ref, wq_ref, k_ref, v_ref, wo_ref, out_ref,
             xall, part, rs_buf,
             ag_send_sems, ag_recv_sems, rs_send_sems, rs_recv_sems):
        my = lax.axis_index("i")
        left = lax.rem(my - 1 + N_DEV, N_DEV)
        right = lax.rem(my + 1, N_DEV)

        barrier = pltpu.get_barrier_semaphore()
        pl.semaphore_signal(barrier, inc=1, device_id=(left,),
                            device_id_type=pl.DeviceIdType.MESH)
        pl.semaphore_signal(barrier, inc=1, device_id=(right,),
                            device_id_type=pl.DeviceIdType.MESH)
        pl.semaphore_wait(barrier, 2)

        xall[my] = x_ref[...]
        for h in range(N_DEV - 1):
            src_chunk = lax.rem(my - h + N_DEV, N_DEV)
            dst_chunk = src_chunk
            rdma = pltpu.make_async_remote_copy(
                src_ref=xall.at[src_chunk],
                dst_ref=xall.at[dst_chunk],
                send_sem=ag_send_sems.at[h],
                recv_sem=ag_recv_sems.at[h],
                device_id=(right,),
                device_id_type=pl.DeviceIdType.MESH,
            )
            rdma.start()
            rdma.wait()

        X = xall[...].reshape(N_DEV * B_PER * SQ, DM)
        Q = jnp.dot(X, wq_ref[...],
                    preferred_element_type=jnp.float32)
        Qb = Q.astype(jnp.bfloat16)

        qi = lax.broadcasted_iota(jnp.int32, (SQ, SQ), 0) // 64
        kj = lax.broadcasted_iota(jnp.int32, (SQ, SQ), 1) // 64
        mask = (qi == kj)[None, :, :]

        acc = None
        for h in range(H_PER):
            qh = Qb[:, h * DH:(h + 1) * DH].reshape(16, SQ, DH)
            scores = lax.dot_general(
                qh, k_ref[h],
                dimension_numbers=(((2,), (2,)), ((0,), (0,))),
                preferred_element_type=jnp.float32,
            ) * 0.125
            scores = jnp.where(mask, scores, -1e9)
            m = jnp.max(scores, axis=-1, keepdims=True)
            w = jnp.exp(scores - m)
            w = (w / jnp.sum(w, axis=-1, keepdims=True)).astype(jnp.bfloat16)
            ctx = lax.dot_general(
                w, v_ref[h],
                dimension_numbers=(((2,), (1,)), ((0,), (0,))),
                preferred_element_type=jnp.float32,
            ).astype(jnp.bfloat16)
            ph = jnp.dot(ctx.reshape(16 * SQ, DH),
                         wo_ref[h * DH:(h + 1) * DH, :],
                         preferred_element_type=jnp.float32)
            acc = ph if acc is None else acc + ph
        part[...] = acc.reshape(N_DEV, B_PER, SQ, DM)

        first = lax.rem(my - 1 + N_DEV, N_DEV)
        rs_buf[0] = part[first]
        for t in range(N_DEV - 1):
            rdma = pltpu.make_async_remote_copy(
                src_ref=rs_buf.at[t],
                dst_ref=rs_buf.at[t + 1],
                send_sem=rs_send_sems.at[t],
                recv_sem=rs_recv_sems.at[t],
                device_id=(right,),
                device_id_type=pl.DeviceIdType.MESH,
            )
            rdma.start()
            rdma.wait()
            chunk = lax.rem(my - 2 - t + N_DEV, N_DEV)
            if t < N_DEV - 2:
                rs_buf[t + 1] = rs_buf[t + 1] + part[chunk]
            else:
                out_ref[...] = rs_buf[t + 1] + part[chunk]

    grid_spec = pltpu.PrefetchScalarGridSpec(
        num_scalar_prefetch=0,
        in_specs=[
            pl.BlockSpec(memory_space=pltpu.VMEM),
            pl.BlockSpec(memory_space=pltpu.VMEM),
            pl.BlockSpec(memory_space=pltpu.VMEM),
            pl.BlockSpec(memory_space=pltpu.VMEM),
            pl.BlockSpec(memory_space=pltpu.VMEM),
        ],
        out_specs=pl.BlockSpec(memory_space=pltpu.VMEM),
        scratch_shapes=[
            pltpu.VMEM((N_DEV, B_PER, SQ, DM), jnp.bfloat16),
            pltpu.VMEM((N_DEV, B_PER, SQ, DM), jnp.float32),
            pltpu.VMEM((N_DEV, B_PER, SQ, DM), jnp.float32),
            pltpu.SemaphoreType.DMA((N_DEV - 1,)),
            pltpu.SemaphoreType.DMA((N_DEV - 1,)),
            pltpu.SemaphoreType.DMA((N_DEV - 1,)),
            pltpu.SemaphoreType.DMA((N_DEV - 1,)),
        ],
    )

    return pl.pallas_call(
        body,
        out_shape=jax.ShapeDtypeStruct((B_PER, SQ, DM), jnp.float32),
        grid_spec=grid_spec,
        compiler_params=pltpu.CompilerParams(
            collective_id=0,
            vmem_limit_bytes=100 * 1024 * 1024,
        ),
    )(x16, wq16, k16, v16, wo16)


# device time: 108628 ns/iter; 1.6850x vs baseline; 1.6850x over previous
import jax
import jax.numpy as jnp
from jax import lax
from jax.experimental import pallas as pl
from jax.experimental.pallas import tpu as pltpu

N_DEV = 8
B_PER = 2
SQ = 256
H_PER = 4
DH = 64
DM = 512


def kernel(x, Wq, K_ext, V_ext, Wo):
    p = lax.axis_index("i")

    k_l = lax.dynamic_slice(K_ext, (0, 0, 4 * p, 0), (16, SQ, H_PER, DH))
    v_l = lax.dynamic_slice(V_ext, (0, 0, 4 * p, 0), (16, SQ, H_PER, DH))
    x16 = x.astype(jnp.bfloat16)
    wq16 = Wq.astype(jnp.bfloat16)
    wo16 = Wo.astype(jnp.bfloat16)
    k16 = jnp.transpose(k_l, (2, 0, 1, 3)).astype(jnp.bfloat16)
    v16 = jnp.transpose(v_l, (2, 0, 1, 3)).astype(jnp.bfloat16)

    def body(x_ref, wq_ref, k_ref, v_ref, wo_ref, out_ref,
             xall, own_part, rs_buf,
             ag_send_sems, ag_recv_sems, rs_send_sems, rs_recv_sems):
        my = lax.axis_index("i")
        left = lax.rem(my - 1 + N_DEV, N_DEV)
        right = lax.rem(my + 1, N_DEV)

        qi = lax.broadcasted_iota(jnp.int32, (SQ, SQ), 0) // 64
        kj = lax.broadcasted_iota(jnp.int32, (SQ, SQ), 1) // 64
        mask = (qi == kj)[None, :, :]

        def partial_for(chunk):
            xc = xall[chunk].reshape(B_PER * SQ, DM)
            Q = jnp.dot(xc, wq_ref[...],
                        preferred_element_type=jnp.float32)
            Qb = Q.astype(jnp.bfloat16)
            acc = None
            for h in range(H_PER):
                kh = k_ref[h, pl.ds(B_PER * chunk, B_PER)]
                vh = v_ref[h, pl.ds(B_PER * chunk, B_PER)]
                qh = Qb[:, h * DH:(h + 1) * DH].reshape(B_PER, SQ, DH)
                scores = lax.dot_general(
                    qh, kh,
                    dimension_numbers=(((2,), (2,)), ((0,), (0,))),
                    preferred_element_type=jnp.float32,
                ) * 0.125
                scores = jnp.where(mask, scores, -1e9)
                m = jnp.max(scores, axis=-1, keepdims=True)
                w = jnp.exp(scores - m)
                w = (w / jnp.sum(w, axis=-1, keepdims=True)
                     ).astype(jnp.bfloat16)
                ctx = lax.dot_general(
                    w, vh,
                    dimension_numbers=(((2,), (1,)), ((0,), (0,))),
                    preferred_element_type=jnp.float32,
                ).astype(jnp.bfloat16)
                ph = jnp.dot(ctx.reshape(B_PER * SQ, DH),
                             wo_ref[h * DH:(h + 1) * DH, :],
                             preferred_element_type=jnp.float32)
                acc = ph if acc is None else acc + ph
            return acc.reshape(B_PER, SQ, DM)

        barrier = pltpu.get_barrier_semaphore()
        pl.semaphore_signal(barrier, inc=1, device_id=(left,),
                            device_id_type=pl.DeviceIdType.MESH)
        pl.semaphore_signal(barrier, inc=1, device_id=(right,),
                            device_id_type=pl.DeviceIdType.MESH)
        pl.semaphore_wait(barrier, 2)

        sends = []

        def ag_send(hop, chunk):
            rdma = pltpu.make_async_remote_copy(
                src_ref=xall.at[chunk],
                dst_ref=xall.at[chunk],
                send_sem=ag_send_sems.at[hop],
                recv_sem=ag_recv_sems.at[hop],
                device_id=(right,),
                device_id_type=pl.DeviceIdType.MESH,
            )
            rdma.start()
            sends.append(rdma)
            return rdma

        def rs_send(hop):
            rdma = pltpu.make_async_remote_copy(
                src_ref=rs_buf.at[hop],
                dst_ref=rs_buf.at[hop + 1],
                send_sem=rs_send_sems.at[hop],
                recv_sem=rs_recv_sems.at[hop],
                device_id=(right,),
                device_id_type=pl.DeviceIdType.MESH,
            )
            rdma.start()
            sends.append(rdma)
            return rdma

        xall[my] = x_ref[...]
        ag_rdmas = [ag_send(0, my)]
        own_part[...] = partial_for(my)

        rs_rdmas = []
        for t in range(N_DEV - 1):
            c = lax.rem(my - 1 - t + N_DEV, N_DEV)
            ag_rdmas[t].wait_recv()
            if t < N_DEV - 2:
                ag_rdmas.append(ag_send(t + 1, c))
            pp = partial_for(c)
            if t == 0:
                rs_buf[0] = pp.astype(jnp.bfloat16)
            else:
                rs_rdmas[t - 1].wait_recv()
                rs_buf[t] = (rs_buf[t].astype(jnp.float32) + pp
                             ).astype(jnp.bfloat16)
            rs_rdmas.append(rs_send(t))

        rs_rdmas[N_DEV - 2].wait_recv()
        out_ref[...] = (rs_buf[N_DEV - 1].astype(jnp.float32)
                        + own_part[...])

        for rdma in sends:
            rdma.wait_send()

    grid_spec = pltpu.PrefetchScalarGridSpec(
        num_scalar_prefetch=0,
        in_specs=[
            pl.BlockSpec(memory_space=pltpu.VMEM),
            pl.BlockSpec(memory_space=pltpu.VMEM),
            pl.BlockSpec(memory_space=pltpu.VMEM),
            pl.BlockSpec(memory_space=pltpu.VMEM),
            pl.BlockSpec(memory_space=pltpu.VMEM),
        ],
        out_specs=pl.BlockSpec(memory_space=pltpu.VMEM),
        scratch_shapes=[
            pltpu.VMEM((N_DEV, B_PER, SQ, DM), jnp.bfloat16),
            pltpu.VMEM((B_PER, SQ, DM), jnp.float32),
            pltpu.VMEM((N_DEV, B_PER, SQ, DM), jnp.bfloat16),
            pltpu.SemaphoreType.DMA((N_DEV - 1,)),
            pltpu.SemaphoreType.DMA((N_DEV - 1,)),
            pltpu.SemaphoreType.DMA((N_DEV - 1,)),
            pltpu.SemaphoreType.DMA((N_DEV - 1,)),
        ],
    )

    return pl.pallas_call(
        body,
        out_shape=jax.ShapeDtypeStruct((B_PER, SQ, DM), jnp.float32),
        grid_spec=grid_spec,
        compiler_params=pltpu.CompilerParams(
            collective_id=0,
            vmem_limit_bytes=100 * 1024 * 1024,
        ),
    )(x16, wq16, k16, v16, wo16)


# device time: 56305 ns/iter; 3.2508x vs baseline; 1.9293x over previous
import jax
import jax.numpy as jnp
from jax import lax
from jax.experimental import pallas as pl
from jax.experimental.pallas import tpu as pltpu

N_DEV = 8
B_PER = 2
SQ = 256
H_PER = 4
DH = 64
DM = 512
DQ = H_PER * DH

R_HOPS = 4
L_HOPS = 3


def kernel(x, Wq, K_ext, V_ext, Wo):
    p = lax.axis_index("i")

    k_l = lax.dynamic_slice(K_ext, (2 * p, 0, 0, 0), (B_PER, SQ, 32, DH))
    v_l = lax.dynamic_slice(V_ext, (2 * p, 0, 0, 0), (B_PER, SQ, 32, DH))
    k16 = jnp.transpose(k_l, (2, 0, 1, 3)).astype(jnp.bfloat16)
    v16 = jnp.transpose(v_l, (2, 0, 1, 3)).astype(jnp.bfloat16)
    x16 = x.astype(jnp.bfloat16)
    wq16 = Wq.astype(jnp.bfloat16)
    wo16 = Wo.astype(jnp.bfloat16)

    def body(x_ref, wq_ref, k_ref, v_ref, wo_ref, out_ref,
             wq_all, wo_all,
             r_send_sems, r_recv_sems, l_send_sems, l_recv_sems):
        my = lax.axis_index("i")
        left = lax.rem(my - 1 + N_DEV, N_DEV)
        right = lax.rem(my + 1, N_DEV)

        qi = lax.broadcasted_iota(jnp.int32, (SQ, SQ), 0) // 64
        kj = lax.broadcasted_iota(jnp.int32, (SQ, SQ), 1) // 64
        mask = (qi == kj)[None, :, :]

        xb = x_ref[...].reshape(B_PER * SQ, DM)

        def add_chunk(c, acc):
            Q = jnp.dot(xb, wq_all[c],
                        preferred_element_type=jnp.float32)
            Qb = Q.astype(jnp.bfloat16)
            for h in range(H_PER):
                kh = k_ref[H_PER * c + h]
                vh = v_ref[H_PER * c + h]
                qh = Qb[:, h * DH:(h + 1) * DH].reshape(B_PER, SQ, DH)
                scores = lax.dot_general(
                    qh, kh,
                    dimension_numbers=(((2,), (2,)), ((0,), (0,))),
                    preferred_element_type=jnp.float32,
                ) * 0.125
                scores = jnp.where(mask, scores, -1e9)
                m = jnp.max(scores, axis=-1, keepdims=True)
                w = jnp.exp(scores - m)
                w = (w / jnp.sum(w, axis=-1, keepdims=True)
                     ).astype(jnp.bfloat16)
                ctx = lax.dot_general(
                    w, vh,
                    dimension_numbers=(((2,), (1,)), ((0,), (0,))),
                    preferred_element_type=jnp.float32,
                ).astype(jnp.bfloat16)
                ph = jnp.dot(ctx.reshape(B_PER * SQ, DH),
                             wo_all[c, h * DH:(h + 1) * DH, :],
                             preferred_element_type=jnp.float32)
                acc = acc + ph
            return acc

        barrier = pltpu.get_barrier_semaphore()
        pl.semaphore_signal(barrier, inc=1, device_id=(left,),
                            device_id_type=pl.DeviceIdType.MESH)
        pl.semaphore_signal(barrier, inc=1, device_id=(right,),
                            device_id_type=pl.DeviceIdType.MESH)
        pl.semaphore_wait(barrier, 2)

        sends = []

        def send_pair(chunk, hop, to, send_sems, recv_sems):
            for buf, s in ((wq_all, 0), (wo_all, 1)):
                rdma = pltpu.make_async_remote_copy(
                    src_ref=buf.at[chunk],
                    dst_ref=buf.at[chunk],
                    send_sem=send_sems.at[2 * hop + s],
                    recv_sem=recv_sems.at[2 * hop + s],
                    device_id=(to,),
                    device_id_type=pl.DeviceIdType.MESH,
                )
                rdma.start()
                sends.append(rdma)
            return sends[-2:]

        def wait_pair(pair):
            pair[0].wait_recv()
            pair[1].wait_recv()

        wq_all[my] = wq_ref[...]
        wo_all[my] = wo_ref[...]
        r_hops = [send_pair(my, 0, right, r_send_sems, r_recv_sems)]
        l_hops = [send_pair(my, 0, left, l_send_sems, l_recv_sems)]
        acc = add_chunk(my, jnp.zeros((B_PER * SQ, DM), jnp.float32))

        for s in range(R_HOPS):
            cr = lax.rem(my - 1 - s + N_DEV, N_DEV)
            wait_pair(r_hops[s])
            if s + 1 < R_HOPS:
                r_hops.append(
                    send_pair(cr, s + 1, right, r_send_sems, r_recv_sems))
            if s < L_HOPS:
                cl = lax.rem(my + 1 + s, N_DEV)
                wait_pair(l_hops[s])
                if s + 1 < L_HOPS:
                    l_hops.append(
                        send_pair(cl, s + 1, left, l_send_sems, l_recv_sems))
                acc = add_chunk(cr, acc)
                acc = add_chunk(cl, acc)
            else:
                acc = add_chunk(cr, acc)

        out_ref[...] = acc.reshape(B_PER, SQ, DM)

        for rdma in sends:
            rdma.wait_send()

    grid_spec = pltpu.PrefetchScalarGridSpec(
        num_scalar_prefetch=0,
        in_specs=[
            pl.BlockSpec(memory_space=pltpu.VMEM),
            pl.BlockSpec(memory_space=pltpu.VMEM),
            pl.BlockSpec(memory_space=pltpu.VMEM),
            pl.BlockSpec(memory_space=pltpu.VMEM),
            pl.BlockSpec(memory_space=pltpu.VMEM),
        ],
        out_specs=pl.BlockSpec(memory_space=pltpu.VMEM),
        scratch_shapes=[
            pltpu.VMEM((N_DEV, DM, DQ), jnp.bfloat16),
            pltpu.VMEM((N_DEV, DQ, DM), jnp.bfloat16),
            pltpu.SemaphoreType.DMA((2 * R_HOPS,)),
            pltpu.SemaphoreType.DMA((2 * R_HOPS,)),
            pltpu.SemaphoreType.DMA((2 * L_HOPS,)),
            pltpu.SemaphoreType.DMA((2 * L_HOPS,)),
        ],
    )

    return pl.pallas_call(
        body,
        out_shape=jax.ShapeDtypeStruct((B_PER, SQ, DM), jnp.float32),
        grid_spec=grid_spec,
        compiler_params=pltpu.CompilerParams(
            collective_id=0,
            vmem_limit_bytes=100 * 1024 * 1024,
        ),
    )(x16, wq16, k16, v16, wo16)
